# baseline (speedup 1.0000x reference)
import functools

import jax
import jax.numpy as jnp
from jax import lax
from jax.experimental import pallas as pl
from jax.experimental.pallas import tpu as pltpu
from jax.experimental.pallas import tpu_sc as plsc


def _down_body(x_ref, a_ref, wg_ref, p_ref, gt_ref):
    x = x_ref[...]
    p_ref[...] = jnp.dot(x, a_ref[...], preferred_element_type=jnp.float32)
    g = jnp.dot(x, wg_ref[...], preferred_element_type=jnp.float32)
    gt_ref[...] = g.T


def _up_body(p_ref, wt_ref, e8_ref, b_ref, o_ref):
    wrow_t = wt_ref[...]
    wfull = lax.dot_general(wrow_t, e8_ref[...],
                            (((0,), (0,)), ((), ())),
                            preferred_element_type=jnp.float32)
    o_ref[...] = jnp.dot(p_ref[...] * wfull, b_ref[...],
                         preferred_element_type=jnp.float32)


def _make_route(T, E):
    info = plsc.get_sparse_core_info()
    NC, NS, L = info.num_cores, info.num_subcores, info.num_lanes
    NW = NC * NS
    per_w = T // NW
    n_chunks = per_w // L
    mesh = plsc.VectorSubcoreMesh(core_axis_name="c", subcore_axis_name="s")

    @functools.partial(
        pl.kernel, mesh=mesh,
        out_type=jax.ShapeDtypeStruct((E, T), jnp.float32),
        scratch_types=[
            pltpu.VMEM((E, per_w), jnp.float32),
            pltpu.VMEM((E, per_w), jnp.float32),
        ],
    )
    def route(gt_hbm, wt_hbm, g_v, w_v):
        wid = lax.axis_index("s") * NC + lax.axis_index("c")
        base = wid * per_w
        pltpu.sync_copy(gt_hbm.at[:, pl.ds(base, per_w)], g_v)
        neg = jnp.float32(-1e30)
        for c in range(n_chunks):
            sl = pl.ds(c * L, L)
            ge = [g_v[e, sl] for e in range(E)]
            m1 = ge[0]
            for e in range(1, E):
                m1 = jnp.maximum(m1, ge[e])
            is1 = [ge[e] == m1 for e in range(E)]
            g2 = [jnp.where(is1[e], neg, ge[e]) for e in range(E)]
            m2 = g2[0]
            for e in range(1, E):
                m2 = jnp.maximum(m2, g2[e])
            t = jnp.exp(m2 - m1)
            w1 = 1.0 / (1.0 + t)
            w2 = t / (1.0 + t)
            for e in range(E):
                w_v[e, sl] = (jnp.where(is1[e], w1, 0.0)
                              + jnp.where(g2[e] == m2, w2, 0.0))
        pltpu.sync_copy(w_v, wt_hbm.at[:, pl.ds(base, per_w)])

    return route


def kernel(inputs, Wg, A, Bm):
    Bsz, S, D = inputs.shape
    E, _, R = A.shape
    T = Bsz * S
    x = inputs.reshape(T, D)
    a_all = jnp.transpose(A, (1, 0, 2)).reshape(D, E * R)
    b_all = Bm.reshape(E * R, D)
    e8 = (jax.lax.broadcasted_iota(jnp.int32, (E, E * R), 1) // R
          == jax.lax.broadcasted_iota(jnp.int32, (E, E * R), 0)
          ).astype(jnp.float32)

    TB = 1024
    p, gt = pl.pallas_call(
        _down_body,
        grid=(T // TB,),
        in_specs=[
            pl.BlockSpec((TB, D), lambda i: (i, 0)),
            pl.BlockSpec((D, E * R), lambda i: (0, 0)),
            pl.BlockSpec((D, E), lambda i: (0, 0)),
        ],
        out_specs=[
            pl.BlockSpec((TB, E * R), lambda i: (i, 0)),
            pl.BlockSpec((E, TB), lambda i: (0, i)),
        ],
        out_shape=[
            jax.ShapeDtypeStruct((T, E * R), jnp.float32),
            jax.ShapeDtypeStruct((E, T), jnp.float32),
        ],
        compiler_params=pltpu.CompilerParams(
            dimension_semantics=("parallel",)),
    )(x, a_all, Wg)

    wrow_t = _make_route(T, E)(gt)

    out = pl.pallas_call(
        _up_body,
        grid=(T // TB,),
        in_specs=[
            pl.BlockSpec((TB, E * R), lambda i: (i, 0)),
            pl.BlockSpec((E, TB), lambda i: (0, i)),
            pl.BlockSpec((E, E * R), lambda i: (0, 0)),
            pl.BlockSpec((E * R, D), lambda i: (0, 0)),
        ],
        out_specs=pl.BlockSpec((TB, D), lambda i: (i, 0)),
        out_shape=jax.ShapeDtypeStruct((T, D), jnp.float32),
        compiler_params=pltpu.CompilerParams(
            dimension_semantics=("parallel",)),
    )(p, wrow_t, e8, b_all)
    return out.reshape(Bsz, S, D)


# speedup vs baseline: 1.3402x; 1.3402x over previous
import functools

import jax
import jax.numpy as jnp
from jax.experimental import pallas as pl
from jax.experimental.pallas import tpu as pltpu


def _body(E, R, x_ref, wg_ref, a_ref, b_ref, e8_ref, o_ref):
    x = x_ref[...]
    g = jnp.dot(x, wg_ref[...], preferred_element_type=jnp.float32)
    neg = jnp.float32(-1e30)
    m1 = jnp.max(g, axis=1, keepdims=True)
    is1 = g == m1
    g2 = jnp.where(is1, neg, g)
    m2 = jnp.max(g2, axis=1, keepdims=True)
    is2 = g2 == m2
    t = jnp.exp(m2 - m1)
    w1 = 1.0 / (1.0 + t)
    w2 = 1.0 - w1
    wrow = jnp.where(is1, w1, jnp.where(is2, w2, 0.0))

    p = jnp.dot(x, a_ref[...], preferred_element_type=jnp.float32)
    wfull = jnp.dot(wrow, e8_ref[...], preferred_element_type=jnp.float32)
    o_ref[...] = jnp.dot(p * wfull, b_ref[...],
                         preferred_element_type=jnp.float32)


def kernel(inputs, Wg, A, Bm):
    Bsz, S, D = inputs.shape
    E, _, R = A.shape
    T = Bsz * S
    x = inputs.reshape(T, D)
    a_all = jnp.transpose(A, (1, 0, 2)).reshape(D, E * R)
    b_all = Bm.reshape(E * R, D)
    e8 = (jax.lax.broadcasted_iota(jnp.int32, (E, E * R), 1) // R
          == jax.lax.broadcasted_iota(jnp.int32, (E, E * R), 0)
          ).astype(jnp.float32)

    TB = 1024
    out = pl.pallas_call(
        functools.partial(_body, E, R),
        grid=(T // TB,),
        in_specs=[
            pl.BlockSpec((TB, D), lambda i: (i, 0)),
            pl.BlockSpec((D, E), lambda i: (0, 0)),
            pl.BlockSpec((D, E * R), lambda i: (0, 0)),
            pl.BlockSpec((E * R, D), lambda i: (0, 0)),
            pl.BlockSpec((E, E * R), lambda i: (0, 0)),
        ],
        out_specs=pl.BlockSpec((TB, D), lambda i: (i, 0)),
        out_shape=jax.ShapeDtypeStruct((T, D), jnp.float32),
        compiler_params=pltpu.CompilerParams(
            dimension_semantics=("parallel",)),
    )(x, Wg, a_all, b_all, e8)
    return out.reshape(Bsz, S, D)
